# Initial kernel scaffold; baseline (speedup 1.0000x reference)
import jax
import jax.numpy as jnp
from jax.experimental import pallas as pl


def kernel(x, table, pe):
    raise NotImplementedError("write your pallas kernel here")


# speedup vs baseline: 1.4304x; 1.4304x over previous
import functools
import math

import jax
import jax.numpy as jnp
from jax import lax
from jax.experimental import pallas as pl
from jax.experimental.pallas import tpu as pltpu
from jax.experimental.pallas import tpu_sc as plsc

_EMBED_DIM = 32
_SEQ_LEN = 200
_BATCH = 4096
_B = _BATCH * _SEQ_LEN
_NW = 32
_B_PER_W = _B // _NW
_CHUNK = 1600
_N_CHUNKS = _B_PER_W // _CHUNK
_REPS = _CHUNK // _SEQ_LEN
_SCALE = math.sqrt(_EMBED_DIM)
_H = _EMBED_DIM // 2


@jax.jit
def _tok_embed(x_flat, table, pe_s):
    mesh = plsc.VectorSubcoreMesh(core_axis_name="c", subcore_axis_name="s")

    @functools.partial(
        pl.kernel,
        mesh=mesh,
        compiler_params=pltpu.CompilerParams(use_tc_tiling_on_sc=False),
        out_type=jax.ShapeDtypeStruct((_B, _EMBED_DIM), jnp.float32),
        scratch_types=[
            pltpu.VMEM((_CHUNK,), jnp.int32),
            pltpu.VMEM((_CHUNK, _EMBED_DIM), jnp.float32),
            pltpu.VMEM((_SEQ_LEN, _EMBED_DIM), jnp.float32),
            pltpu.SemaphoreType.DMA,
        ],
    )
    def k(x_hbm, table_hbm, pe_hbm, out_hbm, idx_v, rows_v, pe_v, sem):
        wid = lax.axis_index("s") * 2 + lax.axis_index("c")
        base = wid * _B_PER_W
        pltpu.sync_copy(pe_hbm, pe_v)

        def chunk_body(g, carry):
            off = base + g * _CHUNK
            pltpu.sync_copy(x_hbm.at[pl.ds(off, _CHUNK)], idx_v)
            pltpu.async_copy(table_hbm.at[idx_v], rows_v, sem).wait()

            def p_body(p, c2):
                pe_lo = pe_v[p, pl.ds(0, _H)]
                pe_hi = pe_v[p, pl.ds(_H, _H)]

                def rep_body(rep, c3):
                    r = rep * _SEQ_LEN + p
                    rows_v[r, pl.ds(0, _H)] = (
                        rows_v[r, pl.ds(0, _H)] * _SCALE + pe_lo
                    )
                    rows_v[r, pl.ds(_H, _H)] = (
                        rows_v[r, pl.ds(_H, _H)] * _SCALE + pe_hi
                    )
                    return c3

                return lax.fori_loop(0, _REPS, rep_body, c2)

            lax.fori_loop(0, _SEQ_LEN, p_body, carry)
            pltpu.sync_copy(rows_v, out_hbm.at[pl.ds(off, _CHUNK), :])
            return carry

        lax.fori_loop(0, _N_CHUNKS, chunk_body, 0)

    return k(x_flat, table, pe_s)


def kernel(x, table, pe):
    x_flat = x.reshape(-1)
    pe_s = pe[0, : x.shape[1], :]
    out = _tok_embed(x_flat, table, pe_s)
    return out.reshape(x.shape[0], x.shape[1], _EMBED_DIM)
